# baseline (speedup 1.0000x reference)
import functools

import jax
import jax.numpy as jnp
from jax import lax
from jax.experimental import pallas as pl
from jax.experimental.pallas import tpu as pltpu
from jax.experimental.pallas import tpu_sc as plsc

_INFO = plsc.get_sparse_core_info()
NW = _INFO.num_cores * _INFO.num_subcores
LANES = 16


def _wid():
    return lax.axis_index("s") * _INFO.num_cores + lax.axis_index("c")


@functools.lru_cache(maxsize=None)
def _build_widen(vocab: int, dim: int):
    assert dim == 64 and vocab % 2 == 0
    full = vocab // 128
    tailw = vocab - full * 128
    base, extra = divmod(full, NW)

    mesh = plsc.VectorSubcoreMesh(core_axis_name="c", subcore_axis_name="s")

    @functools.partial(
        pl.kernel,
        mesh=mesh,
        out_type=jax.ShapeDtypeStruct((vocab // 2, 128), jnp.float32),
        scratch_types=[
            pltpu.VMEM((4, dim, 129), jnp.float32),
            pltpu.VMEM((3, dim, 128), jnp.float32),
            pltpu.SemaphoreType.DMA,
            pltpu.SemaphoreType.DMA,
        ],
        compiler_params=pltpu.CompilerParams(needs_layout_passes=False),
    )
    def widen_kernel(emb_t, tail, table, ibuf, obuf, rsem, wsem):
        w = _wid()
        kcount = jnp.where(w < extra, base + 1, base)
        iota = jax.lax.iota(jnp.int32, LANES)

        def tc_of(k):
            return w + k * NW

        def start_read(k, buf):
            col = pl.multiple_of(tc_of(k) * 128, 128)
            pltpu.async_copy(
                emb_t.at[:, pl.ds(col, 128)], ibuf.at[buf, :, pl.ds(0, 128)], rsem
            )

        def wait_read(k, buf):
            col = pl.multiple_of(tc_of(k) * 128, 128)
            pltpu.make_async_copy(
                emb_t.at[:, pl.ds(col, 128)], ibuf.at[buf, :, pl.ds(0, 128)], rsem
            ).wait()

        def start_write(k, buf):
            row = pl.multiple_of(tc_of(k) * 64, 64)
            pltpu.async_copy(
                obuf.at[buf], table.at[pl.ds(row, 64), :], wsem
            )

        def wait_write():
            pltpu.make_async_copy(
                obuf.at[0], table.at[pl.ds(0, 64), :], wsem
            ).wait()

        def pack(rbuf, wbuf):
            rows4 = tuple(iota + (r * 16) for r in range(4))

            def qbody(q, rows4, rbuf=rbuf, wbuf=wbuf):
                c0 = jnp.full((LANES,), 2 * q, jnp.int32)
                c1 = c0 + 1
                vals = [
                    plsc.load_gather(
                        ibuf.at[rbuf], [rows4[s % 4], c0 if s < 4 else c1]
                    )
                    for s in range(8)
                ]
                for s in range(8):
                    obuf[wbuf, q, pl.ds(s * 16, 16)] = vals[s]
                return rows4

            lax.fori_loop(0, 64, qbody, rows4)

        @pl.when(kcount > 0)
        def _():
            for i in range(3):
                @pl.when(i < kcount)
                def _(i=i):
                    start_read(i, i)

            def body(k, carry):
                @pl.when(k + 3 < kcount)
                def _():
                    start_read(k + 3, lax.rem(k + 3, 4))

                wait_read(k, lax.rem(k, 4))

                @pl.when(k >= 2)
                def _():
                    wait_write()

                pack(lax.rem(k, 4), lax.rem(k, 3))
                start_write(k, lax.rem(k, 3))
                return carry

            lax.fori_loop(0, kcount, body, 0)

            @pl.when(kcount >= 2)
            def _():
                wait_write()

            wait_write()

        if tailw:
            trows = tailw // 2
            @pl.when(w == 0)
            def _():
                pltpu.async_copy(
                    tail, ibuf.at[0, pl.ds(0, trows), pl.ds(0, 128)], rsem
                ).wait()
                pltpu.async_copy(
                    ibuf.at[0, pl.ds(0, trows), pl.ds(0, 128)],
                    table.at[pl.ds(full * 64, trows), :],
                    wsem,
                ).wait()

    return widen_kernel


@functools.lru_cache(maxsize=None)
def _build_gather(hist: int, bsz: int, vocab: int, dim: int):
    assert dim == 64 and bsz % (128 * 1) == 0
    nbb = bsz // 128
    assert nbb == NW

    mesh = plsc.VectorSubcoreMesh(core_axis_name="c", subcore_axis_name="s")

    @functools.partial(
        pl.kernel,
        mesh=mesh,
        out_type=jax.ShapeDtypeStruct((hist, dim, bsz), jnp.float32),
        scratch_types=[
            pltpu.VMEM((3, 128), jnp.int32),
            pltpu.VMEM((3, 128), jnp.int32),
            pltpu.VMEM((3, 128), jnp.int32),
            pltpu.VMEM((3, 128, 129), jnp.float32),
            pltpu.VMEM((2, dim, 128), jnp.float32),
            pltpu.SemaphoreType.DMA,
            pltpu.SemaphoreType.DMA,
            pltpu.SemaphoreType.DMA,
        ],
        compiler_params=pltpu.CompilerParams(needs_layout_passes=False),
    )
    def gather_kernel(x_t, table, ot, idxb, rowb, offb, gbuf, obuf,
                      isem, gsem, osem):
        w = _wid()
        iota = jax.lax.iota(jnp.int32, LANES)

        wcol = pl.multiple_of(w * 128, 128)

        def start_idx(t, buf):
            pltpu.async_copy(
                x_t.at[t, pl.ds(wcol, 128)], idxb.at[buf], isem
            )

        def wait_idx(t, buf):
            pltpu.make_async_copy(
                x_t.at[t, pl.ds(wcol, 128)], idxb.at[buf], isem
            ).wait()

        def start_gather(buf):
            pltpu.async_copy(table.at[rowb.at[buf]], gbuf.at[buf, :, pl.ds(0, 128)], gsem)

        def wait_gather(buf):
            pltpu.make_async_copy(
                table.at[rowb.at[buf]], gbuf.at[buf, :, pl.ds(0, 128)], gsem
            ).wait()

        def start_owrite(t, buf):
            pltpu.async_copy(
                obuf.at[buf], ot.at[t, :, pl.ds(wcol, 128)], osem
            )

        def wait_owrite():
            pltpu.make_async_copy(
                obuf.at[0], ot.at[0, :, pl.ds(0, 128)], osem
            ).wait()

        def split(buf):
            for s in range(8):
                v = idxb[buf, pl.ds(s * 16, 16)]
                rowb[buf, pl.ds(s * 16, 16)] = v >> 1
                offb[buf, pl.ds(s * 16, 16)] = (v & 1) << 6

        def transpose(gb, ob):
            state = (
                tuple(iota + (s * 16) for s in range(8)),
                tuple(offb[gb, pl.ds(s * 16, 16)] for s in range(8)),
            )

            def dbody(d, state, gb=gb, ob=ob):
                rows8, par8 = state
                vals = [
                    plsc.load_gather(gbuf.at[gb], [rows8[s], par8[s] + d])
                    for s in range(8)
                ]
                for s in range(8):
                    obuf[ob, d, pl.ds(s * 16, 16)] = vals[s]
                return state

            lax.fori_loop(0, dim, dbody, state)

        start_idx(0, 0)
        start_idx(1, 1)
        wait_idx(0, 0)
        split(0)
        start_gather(0)

        def body(t, carry):
            t0 = lax.rem(t, 3)
            t1 = lax.rem(t + 1, 3)
            t2 = lax.rem(t + 2, 3)
            ob = lax.rem(t, 2)

            @pl.when(t + 2 < hist)
            def _():
                start_idx(t + 2, t2)

            @pl.when(t + 1 < hist)
            def _():
                wait_idx(t + 1, t1)
                split(t1)
                start_gather(t1)

            wait_gather(t0)

            @pl.when(t >= 2)
            def _():
                wait_owrite()

            transpose(t0, ob)
            start_owrite(t, ob)
            return carry

        lax.fori_loop(0, hist, body, 0)
        wait_owrite()
        wait_owrite()

    return gather_kernel


def kernel(x, emb):
    bsz, hist = x.shape
    vocab, dim = emb.shape
    emb_t = emb.T
    x_t = x.astype(jnp.int32).T
    full = vocab // 128
    tail = emb[full * 128:].reshape(-1, 2 * dim)
    table = _build_widen(vocab, dim)(emb_t, tail)
    ot = _build_gather(hist, bsz, vocab, dim)(x_t, table)
    return ot.transpose(2, 0, 1)


# speedup vs baseline: 2.0960x; 2.0960x over previous
import functools

import jax
import jax.numpy as jnp
from jax import lax
from jax.experimental import pallas as pl
from jax.experimental.pallas import tpu as pltpu
from jax.experimental.pallas import tpu_sc as plsc

_INFO = plsc.get_sparse_core_info()
NW = _INFO.num_cores * _INFO.num_subcores
CHUNK = 128


@functools.lru_cache(maxsize=None)
def _build(n_rows: int, dim: int, vocab: int):
    per_w = n_rows // (NW * CHUNK)
    assert per_w * NW * CHUNK == n_rows and per_w >= 3

    mesh = plsc.VectorSubcoreMesh(core_axis_name="c", subcore_axis_name="s")

    @functools.partial(
        pl.kernel,
        mesh=mesh,
        out_type=jax.ShapeDtypeStruct((n_rows, dim), jnp.float32),
        scratch_types=[
            pltpu.VMEM((3, CHUNK), jnp.int32),
            pltpu.VMEM((3, CHUNK, 2 * dim), jnp.float32),
            pltpu.VMEM((2, CHUNK, dim), jnp.float32),
            pltpu.SemaphoreType.DMA,
            pltpu.SemaphoreType.DMA,
            pltpu.SemaphoreType.DMA,
        ],
    )
    def gather_kernel(xr, emb_p, out, idxb, gbuf, obuf, isem, gsem, osem):
        w = lax.axis_index("s") * _INFO.num_cores + lax.axis_index("c")

        def start_idx(t, buf):
            pltpu.async_copy(xr.at[w, t], idxb.at[buf], isem)

        def wait_idx(t, buf):
            pltpu.make_async_copy(xr.at[w, t], idxb.at[buf], isem).wait()

        def start_gather(buf):
            pltpu.async_copy(emb_p.at[idxb.at[buf]], gbuf.at[buf], gsem)

        def wait_gather(buf):
            pltpu.make_async_copy(
                emb_p.at[idxb.at[buf]], gbuf.at[buf], gsem
            ).wait()

        def compact(gb, ob):
            segs = dim // 16

            def rbody(r, carry, gb=gb, ob=ob):
                vals = [
                    gbuf[gb, r, pl.ds(s * 16, 16)] for s in range(segs)
                ]
                for s in range(segs):
                    obuf[ob, r, pl.ds(s * 16, 16)] = vals[s]
                return carry

            lax.fori_loop(0, CHUNK, rbody, 0)

        def start_owrite(t, buf):
            row = pl.multiple_of((w * per_w + t) * CHUNK, CHUNK)
            pltpu.async_copy(
                obuf.at[buf], out.at[pl.ds(row, CHUNK), :], osem
            )

        def wait_owrite():
            pltpu.make_async_copy(
                obuf.at[0], out.at[pl.ds(0, CHUNK), :], osem
            ).wait()

        start_idx(0, 0)
        start_idx(1, 1)
        wait_idx(0, 0)
        start_gather(0)

        def body(t, carry):
            @pl.when(t + 2 < per_w)
            def _():
                start_idx(t + 2, lax.rem(t + 2, 3))

            @pl.when(t >= 2)
            def _():
                wait_owrite()

            @pl.when(t + 1 < per_w)
            def _():
                wait_idx(t + 1, lax.rem(t + 1, 3))
                start_gather(lax.rem(t + 1, 3))

            wait_gather(lax.rem(t, 3))
            compact(lax.rem(t, 3), lax.rem(t, 2))
            start_owrite(t, lax.rem(t, 2))
            return carry

        lax.fori_loop(0, per_w, body, 0)
        wait_owrite()
        wait_owrite()

    return gather_kernel


def kernel(x, emb):
    bsz, hist = x.shape
    vocab, dim = emb.shape
    n_rows = bsz * hist
    emb_p = jnp.pad(emb, ((0, 0), (0, 128 - dim)))
    per_w = n_rows // (NW * CHUNK)
    xr = x.astype(jnp.int32).reshape(NW, per_w, CHUNK)
    out = _build(n_rows, dim, vocab)(xr, emb_p)
    return out.reshape(bsz, hist, dim)
